# Initial kernel scaffold; baseline (speedup 1.0000x reference)
import functools

import jax
import jax.numpy as jnp
from jax import lax
from jax.experimental import pallas as pl
from jax.experimental.pallas import tpu as pltpu
from jax.experimental.pallas import tpu_sc as plsc

N = 10000
D = 128
E = 320000
EPS = 1e-5

NC, NS, L = 2, 16, 16
NW = NC * NS
EPT = E // NW
K = 80
NCH = EPT // K
RPT = N // NS

_mesh = plsc.VectorSubcoreMesh(
    core_axis_name="c", subcore_axis_name="s", num_cores=NC, num_subcores=NS
)


def _seg_body(with_deg, *refs):
    if with_deg:
        (x_hbm, src_hbm, dst_hbm, zeros_hbm, ones_hbm, parts, degp,
         acc_sh, deg_sh, src_v, dst_v, rows_v, ones_v, sem) = refs
    else:
        (x_hbm, src_hbm, dst_hbm, zeros_hbm, parts,
         acc_sh, src_v, dst_v, rows_v, sem) = refs
    c = lax.axis_index("c")
    s = lax.axis_index("s")
    wid = c * NS + s
    r0 = s * RPT

    pltpu.sync_copy(zeros_hbm.at[pl.ds(r0, RPT)], acc_sh.at[pl.ds(r0, RPT)])
    if with_deg:
        pltpu.sync_copy(zeros_hbm.at[pl.ds(r0, RPT), pl.ds(0, 16)],
                        deg_sh.at[pl.ds(r0, RPT)])
        pltpu.sync_copy(ones_hbm, ones_v)

    pltpu.sync_copy(src_hbm.at[pl.ds(wid * NCH, NCH)], src_v)
    pltpu.sync_copy(dst_hbm.at[pl.ds(wid * NCH, NCH)], dst_v)
    plsc.subcore_barrier()

    def chunk(j, carry):
        pltpu.async_copy(x_hbm.at[src_v.at[j]], rows_v, sem).wait()
        pltpu.sync_copy(rows_v, acc_sh.at[dst_v.at[j]], add=True)
        if with_deg:
            pltpu.sync_copy(ones_v, deg_sh.at[dst_v.at[j]], add=True)
        return carry

    lax.fori_loop(0, NCH, chunk, 0)
    plsc.subcore_barrier()

    pltpu.sync_copy(acc_sh.at[pl.ds(r0, RPT)], parts.at[c, pl.ds(r0, RPT)])
    if with_deg:
        pltpu.sync_copy(deg_sh.at[pl.ds(r0, RPT)], degp.at[c, pl.ds(r0, RPT)])


_sc_seg_deg = functools.partial(
    pl.kernel,
    out_type=[
        jax.ShapeDtypeStruct((NC, N, D), jnp.float32),
        jax.ShapeDtypeStruct((NC, N, 16), jnp.float32),
    ],
    mesh=_mesh,
    scratch_types=[
        pltpu.VMEM_SHARED((N, D), jnp.float32),
        pltpu.VMEM_SHARED((N, 16), jnp.float32),
        pltpu.VMEM((NCH, K), jnp.int32),
        pltpu.VMEM((NCH, K), jnp.int32),
        pltpu.VMEM((K, D), jnp.float32),
        pltpu.VMEM((K, 16), jnp.float32),
        pltpu.SemaphoreType.DMA,
    ],
)(functools.partial(_seg_body, True))

_sc_seg = functools.partial(
    pl.kernel,
    out_type=jax.ShapeDtypeStruct((NC, N, D), jnp.float32),
    mesh=_mesh,
    scratch_types=[
        pltpu.VMEM_SHARED((N, D), jnp.float32),
        pltpu.VMEM((NCH, K), jnp.int32),
        pltpu.VMEM((NCH, K), jnp.int32),
        pltpu.VMEM((K, D), jnp.float32),
        pltpu.SemaphoreType.DMA,
    ],
)(functools.partial(_seg_body, False))


def _tc_body(relu, parts_ref, degp_ref, h_ref, wlT_ref, wrT_ref, b_ref, out_ref):
    ssum = parts_ref[0] + parts_ref[1]
    deg = degp_ref[0, :, 0:1] + degp_ref[1, :, 0:1]
    agg = ssum / jnp.maximum(deg, 1.0)
    out = jnp.dot(agg, wlT_ref[...], preferred_element_type=jnp.float32)
    out = out + jnp.dot(h_ref[...], wrT_ref[...], preferred_element_type=jnp.float32)
    out = out + b_ref[...]
    if relu:
        out = jnp.maximum(out, 0.0)
    out_ref[...] = out


_BN = 1000


def _tc_layer(parts, degp, h, wlT, wrT, b, relu):
    grid = (N // _BN,)
    return pl.pallas_call(
        functools.partial(_tc_body, relu),
        grid=grid,
        in_specs=[
            pl.BlockSpec((NC, _BN, D), lambda i: (0, i, 0)),
            pl.BlockSpec((NC, _BN, 16), lambda i: (0, i, 0)),
            pl.BlockSpec((_BN, D), lambda i: (i, 0)),
            pl.BlockSpec((D, D), lambda i: (0, 0)),
            pl.BlockSpec((D, D), lambda i: (0, 0)),
            pl.BlockSpec((1, D), lambda i: (0, 0)),
        ],
        out_specs=pl.BlockSpec((_BN, D), lambda i: (i, 0)),
        out_shape=jax.ShapeDtypeStruct((N, D), jnp.float32),
    )(parts, degp, h, wlT, wrT, b)


def kernel(x, edge_index, Wl1, bl1, Wr1, Wl2, bl2, Wr2, Wl3, bl3, Wr3,
           g1, be1, g2, be2):
    src = edge_index[0].astype(jnp.int32).reshape(NW * NCH, K)
    dst = edge_index[1].astype(jnp.int32).reshape(NW * NCH, K)
    zeros = jnp.zeros((N, D), jnp.float32)
    ones = jnp.ones((K, 16), jnp.float32)

    gs1 = g1 / jnp.sqrt(1.0 + EPS)
    gs2 = g2 / jnp.sqrt(1.0 + EPS)
    wlT1 = Wl1.T * gs1[None, :]
    wrT1 = Wr1.T * gs1[None, :]
    b1 = (bl1 * gs1 + be1)[None, :]
    wlT2 = Wl2.T * gs2[None, :]
    wrT2 = Wr2.T * gs2[None, :]
    b2 = (bl2 * gs2 + be2)[None, :]
    wlT3 = Wl3.T
    wrT3 = Wr3.T
    b3 = bl3[None, :]

    parts, degp = _sc_seg_deg(x, src, dst, zeros, ones)
    h = _tc_layer(parts, degp, x, wlT1, wrT1, b1, True)
    parts = _sc_seg(h, src, dst, zeros)
    h = _tc_layer(parts, degp, h, wlT2, wrT2, b2, True)
    parts = _sc_seg(h, src, dst, zeros)
    h = _tc_layer(parts, degp, h, wlT3, wrT3, b3, False)
    return h


# speedup vs baseline: 7.7942x; 7.7942x over previous
import functools

import jax
import jax.numpy as jnp
from jax import lax
from jax.experimental import pallas as pl
from jax.experimental.pallas import tpu as pltpu
from jax.experimental.pallas import tpu_sc as plsc

N = 10000
D = 128
E = 320000
EPS = 1e-5

NC, NS, L = 2, 16, 16
NW = NC * NS
EPT = E // NW
K = 125
NCH = EPT // K
STG = 40
NST = NCH // STG
NPAD = 10240
RPT = NPAD // NS

_mesh = plsc.VectorSubcoreMesh(
    core_axis_name="c", subcore_axis_name="s", num_cores=NC, num_subcores=NS
)


def _seg_body(x_hbm, src_hbm, dst_hbm, zeros_hbm, parts,
              acc_sh, src_v, dst_v, rows_v, sem):
    c = lax.axis_index("c")
    s = lax.axis_index("s")
    wid = c * NS + s
    r0 = s * RPT

    pltpu.sync_copy(zeros_hbm, acc_sh.at[pl.ds(r0, RPT)])
    plsc.subcore_barrier()

    for p in range(NST):
        base = wid * NCH + p * STG
        pltpu.sync_copy(src_hbm.at[pl.ds(base, STG)], src_v)
        pltpu.sync_copy(dst_hbm.at[pl.ds(base, STG)], dst_v)

        def chunk(j, carry):
            pltpu.async_copy(x_hbm.at[src_v.at[j]], rows_v, sem).wait()
            pltpu.sync_copy(rows_v, acc_sh.at[dst_v.at[j]], add=True)
            return carry

        lax.fori_loop(0, STG, chunk, 0)

    plsc.subcore_barrier()
    pltpu.sync_copy(acc_sh.at[pl.ds(r0, RPT)], parts.at[c, pl.ds(r0, RPT)])


_sc_seg = functools.partial(
    pl.kernel,
    out_type=jax.ShapeDtypeStruct((NC, NPAD, D), jnp.float32),
    mesh=_mesh,
    scratch_types=[
        pltpu.VMEM_SHARED((NPAD, D), jnp.float32),
        pltpu.VMEM((STG, K), jnp.int32),
        pltpu.VMEM((STG, K), jnp.int32),
        pltpu.VMEM((K, D), jnp.float32),
        pltpu.SemaphoreType.DMA,
    ],
)(_seg_body)


def _deg_body(dst_hbm, zeros_hbm, ones_hbm, degp,
              deg_sh, dst_v, ones_v, sem):
    c = lax.axis_index("c")
    s = lax.axis_index("s")
    wid = c * NS + s
    r0 = s * RPT

    pltpu.sync_copy(zeros_hbm, deg_sh.at[pl.ds(r0, RPT)])
    pltpu.sync_copy(ones_hbm, ones_v)
    plsc.subcore_barrier()

    for p in range(NST):
        base = wid * NCH + p * STG
        pltpu.sync_copy(dst_hbm.at[pl.ds(base, STG)], dst_v)

        def chunk(j, carry):
            pltpu.sync_copy(ones_v, deg_sh.at[dst_v.at[j]], add=True)
            return carry

        lax.fori_loop(0, STG, chunk, 0)

    plsc.subcore_barrier()
    pltpu.sync_copy(deg_sh.at[pl.ds(r0, RPT)], degp.at[c, pl.ds(r0, RPT)])


_sc_deg = functools.partial(
    pl.kernel,
    out_type=jax.ShapeDtypeStruct((NC, NPAD, D), jnp.float32),
    mesh=_mesh,
    scratch_types=[
        pltpu.VMEM_SHARED((NPAD, D), jnp.float32),
        pltpu.VMEM((STG, K), jnp.int32),
        pltpu.VMEM((K, D), jnp.float32),
        pltpu.SemaphoreType.DMA,
    ],
)(_deg_body)


def _tc_body(relu, parts_ref, degp_ref, h_ref, wlT_ref, wrT_ref, b_ref, out_ref):
    ssum = parts_ref[0] + parts_ref[1]
    deg = degp_ref[0, :, 0:1] + degp_ref[1, :, 0:1]
    agg = ssum / jnp.maximum(deg, 1.0)
    out = jnp.dot(agg, wlT_ref[...], preferred_element_type=jnp.float32)
    out = out + jnp.dot(h_ref[...], wrT_ref[...], preferred_element_type=jnp.float32)
    out = out + b_ref[...]
    if relu:
        out = jnp.maximum(out, 0.0)
    out_ref[...] = out


_BN = 1000


def _tc_layer(parts, degp, h, wlT, wrT, b, relu):
    grid = (N // _BN,)
    return pl.pallas_call(
        functools.partial(_tc_body, relu),
        grid=grid,
        in_specs=[
            pl.BlockSpec((NC, _BN, D), lambda i: (0, i, 0)),
            pl.BlockSpec((NC, _BN, D), lambda i: (0, i, 0)),
            pl.BlockSpec((_BN, D), lambda i: (i, 0)),
            pl.BlockSpec((D, D), lambda i: (0, 0)),
            pl.BlockSpec((D, D), lambda i: (0, 0)),
            pl.BlockSpec((1, D), lambda i: (0, 0)),
        ],
        out_specs=pl.BlockSpec((_BN, D), lambda i: (i, 0)),
        out_shape=jax.ShapeDtypeStruct((N, D), jnp.float32),
    )(parts, degp, h, wlT, wrT, b)


def kernel(x, edge_index, Wl1, bl1, Wr1, Wl2, bl2, Wr2, Wl3, bl3, Wr3,
           g1, be1, g2, be2):
    src = edge_index[0].astype(jnp.int32).reshape(NW * NCH, K)
    dst = edge_index[1].astype(jnp.int32).reshape(NW * NCH, K)
    zeros = jnp.zeros((RPT, D), jnp.float32)
    ones = jnp.ones((K, D), jnp.float32)

    gs1 = g1 / jnp.sqrt(1.0 + EPS)
    gs2 = g2 / jnp.sqrt(1.0 + EPS)
    wlT1 = Wl1.T * gs1[None, :]
    wrT1 = Wr1.T * gs1[None, :]
    b1 = (bl1 * gs1 + be1)[None, :]
    wlT2 = Wl2.T * gs2[None, :]
    wrT2 = Wr2.T * gs2[None, :]
    b2 = (bl2 * gs2 + be2)[None, :]
    wlT3 = Wl3.T
    wrT3 = Wr3.T
    b3 = bl3[None, :]

    degp = _sc_deg(dst, zeros, ones)
    parts = _sc_seg(x, src, dst, zeros)
    h = _tc_layer(parts, degp, x, wlT1, wrT1, b1, True)
    parts = _sc_seg(h, src, dst, zeros)
    h = _tc_layer(parts, degp, h, wlT2, wrT2, b2, True)
    parts = _sc_seg(h, src, dst, zeros)
    h = _tc_layer(parts, degp, h, wlT3, wrT3, b3, False)
    return h


# speedup vs baseline: 10.9663x; 1.4070x over previous
import functools

import jax
import jax.numpy as jnp
from jax import lax
from jax.experimental import pallas as pl
from jax.experimental.pallas import tpu as pltpu
from jax.experimental.pallas import tpu_sc as plsc

N = 10000
D = 128
E = 320000
EPS = 1e-5

NC, NS, L = 2, 16, 16
NW = NC * NS
EPT = E // NW
K = 125
NCH = EPT // K
STG = 40
NST = NCH // STG
NPAD = 10240
RPT = NPAD // NS

_mesh = plsc.VectorSubcoreMesh(
    core_axis_name="c", subcore_axis_name="s", num_cores=NC, num_subcores=NS
)


def _seg_body(x_hbm, src_hbm, dst_hbm, zeros_hbm, parts,
              acc_sh, src_v, dst_v, rows_a, rows_b, ga, gb):
    c = lax.axis_index("c")
    s = lax.axis_index("s")
    wid = c * NS + s
    r0 = s * RPT

    pltpu.sync_copy(zeros_hbm, acc_sh.at[pl.ds(r0, RPT)])
    plsc.subcore_barrier()

    for p in range(NST):
        base = wid * NCH + p * STG
        pltpu.sync_copy(src_hbm.at[pl.ds(base, STG)], src_v)
        pltpu.sync_copy(dst_hbm.at[pl.ds(base, STG)], dst_v)

        pltpu.async_copy(x_hbm.at[src_v.at[0]], rows_a, ga)

        def chunk(j, carry):
            nxt = j + 1

            @pl.when(j % 2 == 0)
            def _even():
                @pl.when(nxt < STG)
                def _():
                    pltpu.async_copy(x_hbm.at[src_v.at[nxt]], rows_b, gb)
                pltpu.make_async_copy(x_hbm.at[src_v.at[j]], rows_a, ga).wait()
                pltpu.sync_copy(rows_a, acc_sh.at[dst_v.at[j]], add=True)

            @pl.when(j % 2 == 1)
            def _odd():
                @pl.when(nxt < STG)
                def _():
                    pltpu.async_copy(x_hbm.at[src_v.at[nxt]], rows_a, ga)
                pltpu.make_async_copy(x_hbm.at[src_v.at[j]], rows_b, gb).wait()
                pltpu.sync_copy(rows_b, acc_sh.at[dst_v.at[j]], add=True)

            return carry

        lax.fori_loop(0, STG, chunk, 0)

    plsc.subcore_barrier()
    pltpu.sync_copy(acc_sh.at[pl.ds(r0, RPT)], parts.at[c, pl.ds(r0, RPT)])


_sc_seg = functools.partial(
    pl.kernel,
    out_type=jax.ShapeDtypeStruct((NC, NPAD, D), jnp.float32),
    mesh=_mesh,
    scratch_types=[
        pltpu.VMEM_SHARED((NPAD, D), jnp.float32),
        pltpu.VMEM((STG, K), jnp.int32),
        pltpu.VMEM((STG, K), jnp.int32),
        pltpu.VMEM((K, D), jnp.float32),
        pltpu.VMEM((K, D), jnp.float32),
        pltpu.SemaphoreType.DMA,
        pltpu.SemaphoreType.DMA,
    ],
)(_seg_body)


def _deg_body(dst_hbm, zeros_hbm, ones_hbm, degp,
              deg_sh, dst_v, ones_v, sem):
    c = lax.axis_index("c")
    s = lax.axis_index("s")
    wid = c * NS + s
    r0 = s * RPT

    pltpu.sync_copy(zeros_hbm, deg_sh.at[pl.ds(r0, RPT)])
    pltpu.sync_copy(ones_hbm, ones_v)
    plsc.subcore_barrier()

    for p in range(NST):
        base = wid * NCH + p * STG
        pltpu.sync_copy(dst_hbm.at[pl.ds(base, STG)], dst_v)

        def chunk(j, carry):
            pltpu.sync_copy(ones_v, deg_sh.at[dst_v.at[j]], add=True)
            return carry

        lax.fori_loop(0, STG, chunk, 0)

    plsc.subcore_barrier()
    pltpu.sync_copy(deg_sh.at[pl.ds(r0, RPT)], degp.at[c, pl.ds(r0, RPT)])


_sc_deg = functools.partial(
    pl.kernel,
    out_type=jax.ShapeDtypeStruct((NC, NPAD, D), jnp.float32),
    mesh=_mesh,
    scratch_types=[
        pltpu.VMEM_SHARED((NPAD, D), jnp.float32),
        pltpu.VMEM((STG, K), jnp.int32),
        pltpu.VMEM((K, D), jnp.float32),
        pltpu.SemaphoreType.DMA,
    ],
)(_deg_body)


def _tc_body(relu, parts_ref, degp_ref, h_ref, wlT_ref, wrT_ref, b_ref, out_ref):
    ssum = parts_ref[0] + parts_ref[1]
    deg = degp_ref[0, :, 0:1] + degp_ref[1, :, 0:1]
    agg = ssum / jnp.maximum(deg, 1.0)
    out = jnp.dot(agg, wlT_ref[...], preferred_element_type=jnp.float32)
    out = out + jnp.dot(h_ref[...], wrT_ref[...], preferred_element_type=jnp.float32)
    out = out + b_ref[...]
    if relu:
        out = jnp.maximum(out, 0.0)
    out_ref[...] = out


_BN = 1000


def _tc_layer(parts, degp, h, wlT, wrT, b, relu):
    grid = (N // _BN,)
    return pl.pallas_call(
        functools.partial(_tc_body, relu),
        grid=grid,
        in_specs=[
            pl.BlockSpec((NC, _BN, D), lambda i: (0, i, 0)),
            pl.BlockSpec((NC, _BN, D), lambda i: (0, i, 0)),
            pl.BlockSpec((_BN, D), lambda i: (i, 0)),
            pl.BlockSpec((D, D), lambda i: (0, 0)),
            pl.BlockSpec((D, D), lambda i: (0, 0)),
            pl.BlockSpec((1, D), lambda i: (0, 0)),
        ],
        out_specs=pl.BlockSpec((_BN, D), lambda i: (i, 0)),
        out_shape=jax.ShapeDtypeStruct((N, D), jnp.float32),
    )(parts, degp, h, wlT, wrT, b)


def kernel(x, edge_index, Wl1, bl1, Wr1, Wl2, bl2, Wr2, Wl3, bl3, Wr3,
           g1, be1, g2, be2):
    src = edge_index[0].astype(jnp.int32).reshape(NW * NCH, K)
    dst = edge_index[1].astype(jnp.int32).reshape(NW * NCH, K)
    zeros = jnp.zeros((RPT, D), jnp.float32)
    ones = jnp.ones((K, D), jnp.float32)

    gs1 = g1 / jnp.sqrt(1.0 + EPS)
    gs2 = g2 / jnp.sqrt(1.0 + EPS)
    wlT1 = Wl1.T * gs1[None, :]
    wrT1 = Wr1.T * gs1[None, :]
    b1 = (bl1 * gs1 + be1)[None, :]
    wlT2 = Wl2.T * gs2[None, :]
    wrT2 = Wr2.T * gs2[None, :]
    b2 = (bl2 * gs2 + be2)[None, :]
    wlT3 = Wl3.T
    wrT3 = Wr3.T
    b3 = bl3[None, :]

    degp = _sc_deg(dst, zeros, ones)
    parts = _sc_seg(x, src, dst, zeros)
    h = _tc_layer(parts, degp, x, wlT1, wrT1, b1, True)
    parts = _sc_seg(h, src, dst, zeros)
    h = _tc_layer(parts, degp, h, wlT2, wrT2, b2, True)
    parts = _sc_seg(h, src, dst, zeros)
    h = _tc_layer(parts, degp, h, wlT3, wrT3, b3, False)
    return h
